# baseline (speedup 1.0000x reference)
import functools

import jax
import jax.numpy as jnp
from jax import lax
from jax.experimental import pallas as pl
from jax.experimental.pallas import tpu as pltpu
from jax.experimental.pallas import tpu_sc as plsc

B = 320000
D = 128
V = 10000
NW = 32
BPW = B // NW
E = 80
NCH = BPW // E
G = E // 16

_mesh = plsc.VectorSubcoreMesh(core_axis_name="c", subcore_axis_name="s")


@functools.partial(
    pl.kernel,
    mesh=_mesh,
    out_type=jax.ShapeDtypeStruct((B,), jnp.float32),
    scratch_types=[
        pltpu.VMEM((BPW,), jnp.int32),
        pltpu.VMEM((BPW,), jnp.int32),
        pltpu.VMEM((BPW + 16,), jnp.float32),
        pltpu.VMEM((E, D), jnp.bfloat16),
        pltpu.VMEM((E, D), jnp.bfloat16),
        pltpu.VMEM((E, D), jnp.bfloat16),
        pltpu.VMEM((E, D), jnp.bfloat16),
        pltpu.VMEM_SHARED((V, D), jnp.bfloat16),
        pltpu.SemaphoreType.DMA,
        pltpu.SemaphoreType.DMA,
    ],
    compiler_params=pltpu.CompilerParams(
        needs_layout_passes=False, use_tc_tiling_on_sc=False
    ),
)
def _sc_dot(table, edges, o_hbm, sidx, didx, obuf,
            sr0, dr0, sr1, dr1, stab, sem0, sem1):
    sid = lax.axis_index("s")
    wid = sid * 2 + lax.axis_index("c")
    base = wid * BPW
    vps = V // 16
    pltpu.sync_copy(table.at[pl.ds(sid * vps, vps)], stab.at[pl.ds(sid * vps, vps)])
    pltpu.sync_copy(edges.at[pl.ds(base, BPW)], sidx)
    pltpu.sync_copy(edges.at[pl.ds(B + base, BPW)], didx)
    plsc.subcore_barrier()

    bufs = ((sr0, dr0, sem0), (sr1, dr1, sem1))
    last_lane = lax.iota(jnp.int32, 16) == 15

    def issue(ci, b):
        off = ci * E
        srow, drow, sem = bufs[b]
        pltpu.async_copy(stab.at[sidx.at[pl.ds(off, E)]], srow, sem)
        pltpu.async_copy(stab.at[didx.at[pl.ds(off, E)]], drow, sem)

    issue(0, 0)

    def compute(ci, b):
        srow, drow, sem = bufs[b]
        pltpu.make_async_copy(table.at[pl.ds(0, E)], srow, sem).wait()
        pltpu.make_async_copy(table.at[pl.ds(0, E)], drow, sem).wait()
        off = ci * E

        def edge4(k4, carry):
            for u in range(4):
                e = k4 * 4 + u
                prods = []
                for j in range(D // 32):
                    s = srow[e, pl.ds(j * 32, 32)]
                    t = drow[e, pl.ds(j * 32, 32)]
                    prods.append(s * t)
                q01 = prods[0] + prods[1]
                q23 = prods[2] + prods[3]
                a, b2 = plsc.unpack(q01, format=plsc.PackFormat.INTERLEAVED)
                c, d2 = plsc.unpack(q23, format=plsc.PackFormat.INTERLEAVED)
                acc = (a + b2) + (c + d2)
                csum = plsc.cumsum(acc)
                plsc.store_compressed(
                    obuf.at[pl.ds(off + e, 16)], csum, mask=last_lane
                )
            return carry

        lax.fori_loop(0, E // 4, edge4, 0)

    def pair(pi, carry):
        for b in range(2):
            ci = pi * 2 + b

            @pl.when(ci + 1 < NCH)
            def _():
                issue(ci + 1, 1 - b)

            @pl.when(ci < NCH)
            def _():
                compute(ci, b)
        return carry

    lax.fori_loop(0, (NCH + 1) // 2, pair, 0)
    pltpu.sync_copy(obuf.at[pl.ds(0, BPW)], o_hbm.at[pl.ds(base, BPW)])


def kernel(out, edge_label_index):
    edges = edge_label_index.astype(jnp.int32).reshape(-1)
    return _sc_dot(out.astype(jnp.bfloat16), edges)


# speedup vs baseline: 2.3411x; 2.3411x over previous
import functools

import jax
import jax.numpy as jnp
from jax import lax
from jax.experimental import pallas as pl
from jax.experimental.pallas import tpu as pltpu
from jax.experimental.pallas import tpu_sc as plsc

B = 320000
D = 128
V = 10000
NW = 32
BPW = B // NW
E = 80
NCH = BPW // E
G = E // 16

_mesh = plsc.VectorSubcoreMesh(core_axis_name="c", subcore_axis_name="s")


@functools.partial(
    pl.kernel,
    mesh=_mesh,
    out_type=jax.ShapeDtypeStruct((B,), jnp.float32),
    scratch_types=[
        pltpu.VMEM((BPW,), jnp.int32),
        pltpu.VMEM((BPW,), jnp.int32),
        pltpu.VMEM((BPW + 16,), jnp.float32),
        pltpu.VMEM((E, D), jnp.bfloat16),
        pltpu.VMEM((E, D), jnp.bfloat16),
        pltpu.VMEM((E, D), jnp.bfloat16),
        pltpu.VMEM((E, D), jnp.bfloat16),
        pltpu.VMEM_SHARED((V, D), jnp.bfloat16),
        pltpu.SemaphoreType.DMA,
        pltpu.SemaphoreType.DMA,
    ],
    compiler_params=pltpu.CompilerParams(
        needs_layout_passes=False, use_tc_tiling_on_sc=False
    ),
)
def _sc_dot(table, edges, o_hbm, sidx, didx, obuf,
            sr0, dr0, sr1, dr1, stab, sem0, sem1):
    sid = lax.axis_index("s")
    wid = sid * 2 + lax.axis_index("c")
    base = wid * BPW
    vps = V // 16
    pltpu.sync_copy(table.at[pl.ds(sid * vps, vps)], stab.at[pl.ds(sid * vps, vps)])
    pltpu.sync_copy(edges.at[pl.ds(base, BPW)], sidx)
    pltpu.sync_copy(edges.at[pl.ds(B + base, BPW)], didx)
    plsc.subcore_barrier()

    bufs = ((sr0, dr0, sem0), (sr1, dr1, sem1))
    lane = lax.iota(jnp.int32, 16)

    def issue(ci, b):
        off = ci * E
        srow, drow, sem = bufs[b]
        pltpu.async_copy(stab.at[sidx.at[pl.ds(off, E)]], srow, sem)
        pltpu.async_copy(stab.at[didx.at[pl.ds(off, E)]], drow, sem)

    issue(0, 0)

    def compute(ci, b):
        srow, drow, sem = bufs[b]
        pltpu.make_async_copy(table.at[pl.ds(0, E)], srow, sem).wait()
        pltpu.make_async_copy(table.at[pl.ds(0, E)], drow, sem).wait()
        off = ci * E

        def group(g, carry):
            def edge4(k4, out16):
                for u in range(4):
                    k = k4 * 4 + u
                    e = g * 16 + k
                    prods = []
                    for j in range(D // 32):
                        s = srow[e, pl.ds(j * 32, 32)]
                        t = drow[e, pl.ds(j * 32, 32)]
                        prods.append(s * t)
                    q01 = prods[0] + prods[1]
                    q23 = prods[2] + prods[3]
                    a, b2 = plsc.unpack(q01, format=plsc.PackFormat.INTERLEAVED)
                    c, d2 = plsc.unpack(q23, format=plsc.PackFormat.INTERLEAVED)
                    tot = jnp.sum((a + b2) + (c + d2))
                    out16 = jnp.where(lane == k, tot, out16)
                return out16

            out16 = lax.fori_loop(0, 4, edge4, jnp.zeros((16,), jnp.float32))
            obuf[pl.ds(off + g * 16, 16)] = out16
            return carry

        lax.fori_loop(0, G, group, 0)

    def pair(pi, carry):
        for b in range(2):
            ci = pi * 2 + b

            @pl.when(ci + 1 < NCH)
            def _():
                issue(ci + 1, 1 - b)

            @pl.when(ci < NCH)
            def _():
                compute(ci, b)
        return carry

    lax.fori_loop(0, (NCH + 1) // 2, pair, 0)
    pltpu.sync_copy(obuf.at[pl.ds(0, BPW)], o_hbm.at[pl.ds(base, BPW)])


def kernel(out, edge_label_index):
    edges = edge_label_index.astype(jnp.int32).reshape(-1)
    return _sc_dot(out.astype(jnp.bfloat16), edges)


# speedup vs baseline: 2.3797x; 1.0165x over previous
import functools

import jax
import jax.numpy as jnp
from jax import lax
from jax.experimental import pallas as pl
from jax.experimental.pallas import tpu as pltpu
from jax.experimental.pallas import tpu_sc as plsc

B = 320000
D = 128
V = 10000
NW = 32
BPW = B // NW
E = 80
NCH = BPW // E
G = E // 16

_mesh = plsc.VectorSubcoreMesh(core_axis_name="c", subcore_axis_name="s")


@functools.partial(
    pl.kernel,
    mesh=_mesh,
    out_type=jax.ShapeDtypeStruct((B,), jnp.float32),
    scratch_types=[
        pltpu.VMEM((BPW,), jnp.int32),
        pltpu.VMEM((BPW,), jnp.int32),
        pltpu.VMEM((BPW + 16,), jnp.float32),
        pltpu.VMEM((E, D), jnp.bfloat16),
        pltpu.VMEM((E, D), jnp.bfloat16),
        pltpu.VMEM((E, D), jnp.bfloat16),
        pltpu.VMEM((E, D), jnp.bfloat16),
        pltpu.VMEM((E, D), jnp.bfloat16),
        pltpu.VMEM((E, D), jnp.bfloat16),
        pltpu.VMEM((E, D), jnp.bfloat16),
        pltpu.VMEM((E, D), jnp.bfloat16),
        pltpu.VMEM_SHARED((V, D), jnp.bfloat16),
        pltpu.SemaphoreType.DMA,
        pltpu.SemaphoreType.DMA,
        pltpu.SemaphoreType.DMA,
        pltpu.SemaphoreType.DMA,
    ],
    compiler_params=pltpu.CompilerParams(
        needs_layout_passes=False, use_tc_tiling_on_sc=False
    ),
)
def _sc_dot(table, edges, o_hbm, sidx, didx, obuf,
            sr0, dr0, sr1, dr1, sr2, dr2, sr3, dr3, stab,
            sem0, sem1, sem2, sem3):
    sid = lax.axis_index("s")
    wid = sid * 2 + lax.axis_index("c")
    base = wid * BPW
    vps = V // 16
    pltpu.sync_copy(table.at[pl.ds(sid * vps, vps)], stab.at[pl.ds(sid * vps, vps)])
    pltpu.sync_copy(edges.at[pl.ds(base, BPW)], sidx)
    pltpu.sync_copy(edges.at[pl.ds(B + base, BPW)], didx)
    plsc.subcore_barrier()

    bufs = ((sr0, dr0, sem0), (sr1, dr1, sem1),
            (sr2, dr2, sem2), (sr3, dr3, sem3))
    lane = lax.iota(jnp.int32, 16)

    def issue(ci, b):
        off = ci * E
        srow, drow, sem = bufs[b]
        pltpu.async_copy(stab.at[sidx.at[pl.ds(off, E)]], srow, sem)
        pltpu.async_copy(stab.at[didx.at[pl.ds(off, E)]], drow, sem)

    issue(0, 0)
    issue(1, 1)
    issue(2, 2)

    def compute(ci, b):
        srow, drow, sem = bufs[b]
        pltpu.make_async_copy(table.at[pl.ds(0, E)], srow, sem).wait()
        pltpu.make_async_copy(table.at[pl.ds(0, E)], drow, sem).wait()
        off = ci * E

        def group(g, carry):
            def edge4(k4, out16):
                for u in range(4):
                    k = k4 * 4 + u
                    e = g * 16 + k
                    prods = []
                    for j in range(D // 32):
                        s = srow[e, pl.ds(j * 32, 32)]
                        t = drow[e, pl.ds(j * 32, 32)]
                        prods.append(s * t)
                    q01 = prods[0] + prods[1]
                    q23 = prods[2] + prods[3]
                    a, b2 = plsc.unpack(q01, format=plsc.PackFormat.INTERLEAVED)
                    c, d2 = plsc.unpack(q23, format=plsc.PackFormat.INTERLEAVED)
                    tot = jnp.sum((a + b2) + (c + d2))
                    out16 = jnp.where(lane == k, tot, out16)
                return out16

            out16 = lax.fori_loop(0, 4, edge4, jnp.zeros((16,), jnp.float32))
            obuf[pl.ds(off + g * 16, 16)] = out16
            return carry

        lax.fori_loop(0, G, group, 0)

    def quad(qi, carry):
        for b in range(4):
            ci = qi * 4 + b

            @pl.when(ci + 3 < NCH)
            def _():
                issue(ci + 3, (b + 3) % 4)

            @pl.when(ci < NCH)
            def _():
                compute(ci, b)
        return carry

    lax.fori_loop(0, (NCH + 3) // 4, quad, 0)
    pltpu.sync_copy(obuf.at[pl.ds(0, BPW)], o_hbm.at[pl.ds(base, BPW)])


def kernel(out, edge_label_index):
    edges = edge_label_index.astype(jnp.int32).reshape(-1)
    return _sc_dot(out.astype(jnp.bfloat16), edges)
